# baseline (speedup 1.0000x reference)
import functools

import jax
import jax.numpy as jnp
from jax import lax
from jax.experimental import pallas as pl
from jax.experimental.pallas import tpu as pltpu
from jax.experimental.pallas import tpu_sc as plsc

_N = 50000
_F = 32
_N4 = _N // 4
_BLK = 512
_TCGRID = pl.cdiv(_N4, _BLK)

_NW = 32
_NPW = 1568
_NPAD = _NW * _NPW
_N4PAD = _NPAD // 4
_C = 32
_CHUNKS = _NPW // _C
_G = 8
_E = _C * _F


def _prep_body(inp_ref, wbig_ref, bp_ref, m_ref, idx_ref, ewp_ref):
    blk = inp_ref[...]
    m = jnp.maximum(
        jnp.dot(blk, wbig_ref[...], preferred_element_type=jnp.float32)
        + bp_ref[...], 0.0)
    m_ref[...] = m
    nbr = jnp.concatenate(
        [blk[:, 96 * j + 32:96 * j + 64] for j in range(4)], axis=1)
    idx_ref[...] = jnp.clip((nbr * float(_N)).astype(jnp.int32), 0, _N - 1)
    ewp_ref[...] = jnp.concatenate(
        [blk[:, 96 * j + 64:96 * j + 96] for j in range(4)], axis=1)


def _mid_body(inp_ref, a_ref, wua_ref, wub_ref, bu_ref, wp_ref, bp_ref,
              h_ref, m2_ref):
    h = jnp.maximum(
        jnp.dot(inp_ref[...], wua_ref[...], preferred_element_type=jnp.float32)
        + jnp.dot(a_ref[...], wub_ref[...], preferred_element_type=jnp.float32)
        + bu_ref[...], 0.0)
    h_ref[...] = h
    m2_ref[...] = jnp.maximum(
        jnp.dot(h, wp_ref[...], preferred_element_type=jnp.float32)
        + bp_ref[...], 0.0)


def _final_body(h_ref, a_ref, wua_ref, wub_ref, bu_ref, out_ref):
    out_ref[...] = jnp.maximum(
        jnp.dot(h_ref[...], wua_ref[...], preferred_element_type=jnp.float32)
        + jnp.dot(a_ref[...], wub_ref[...], preferred_element_type=jnp.float32)
        + bu_ref[...], 0.0)


def _row_spec():
    return pl.BlockSpec((_BLK, 128), lambda i: (i, 0))


def _inp_spec():
    return pl.BlockSpec((_BLK, 384), lambda i: (i, 0))


def _w_spec(rows=128):
    return pl.BlockSpec((rows, 128), lambda i: (0, 0))


def _b_spec():
    return pl.BlockSpec((1, 128), lambda i: (0, 0))


def _tc_prep(inp384, wbig, bp):
    return pl.pallas_call(
        _prep_body,
        grid=(_TCGRID,),
        in_specs=[_inp_spec(), _w_spec(384), _b_spec()],
        out_specs=[_row_spec(), _row_spec(), _row_spec()],
        out_shape=[
            jax.ShapeDtypeStruct((_N4PAD, 128), jnp.float32),
            jax.ShapeDtypeStruct((_N4PAD, 128), jnp.int32),
            jax.ShapeDtypeStruct((_N4PAD, 128), jnp.float32),
        ],
    )(inp384, wbig, bp)


def _tc_mid(inp384, a4, wua_big, wub, bu, wp, bp):
    return pl.pallas_call(
        _mid_body,
        grid=(_TCGRID,),
        in_specs=[_inp_spec(), _row_spec(), _w_spec(384), _w_spec(), _b_spec(),
                  _w_spec(), _b_spec()],
        out_specs=[_row_spec(), _row_spec()],
        out_shape=[
            jax.ShapeDtypeStruct((_N4, 128), jnp.float32),
            jax.ShapeDtypeStruct((_N4PAD, 128), jnp.float32),
        ],
    )(inp384, a4, wua_big, wub, bu, wp, bp)


def _tc_final(h4, a4, wua, wub, bu):
    return pl.pallas_call(
        _final_body,
        grid=(_TCGRID,),
        in_specs=[_row_spec(), _row_spec(), _w_spec(), _w_spec(), _b_spec()],
        out_specs=_row_spec(),
        out_shape=jax.ShapeDtypeStruct((_N4, 128), jnp.float32),
    )(h4, a4, wua, wub, bu)


@functools.lru_cache(maxsize=1)
def _make_agg():
    mesh = plsc.VectorSubcoreMesh(core_axis_name="c", subcore_axis_name="s")

    @functools.partial(
        pl.kernel,
        mesh=mesh,
        out_type=jax.ShapeDtypeStruct((_NPAD * _F,), jnp.float32),
        scratch_types=[
            pltpu.VMEM((2, _G, 128), jnp.int32),
            pltpu.VMEM((2, 2 * _C, 16), jnp.float32),
            pltpu.VMEM((2, _E, _F), jnp.float32),
            pltpu.VMEM((2, _E), jnp.float32),
            pltpu.SemaphoreType.DMA((2,)),
            pltpu.SemaphoreType.DMA((2,)),
            pltpu.SemaphoreType.DMA((2,)),
            pltpu.SemaphoreType.DMA((2,)),
        ],
        compiler_params=pltpu.CompilerParams(use_tc_tiling_on_sc=False),
    )
    def agg(m_hbm, idx_hbm, ew_hbm, out_hbm, idx_v, ew_v, rows_v, out_v,
            isem, esem, gsem, osem):
        wid = lax.axis_index("s") * 2 + lax.axis_index("c")
        node0_w = wid * _NPW

        def node0_of(g):
            return pl.multiple_of(node0_w + g * _C, _C)

        def issue_idx(g, b):
            node0 = node0_of(g)
            row0 = pl.multiple_of(node0 // 4, _G)
            pltpu.async_copy(idx_hbm.at[pl.ds(row0, _G)], idx_v.at[b],
                             isem.at[b])

        def issue_ew(g, b):
            node0 = node0_of(g)
            pltpu.async_copy(
                ew_hbm.at[pl.ds(pl.multiple_of(2 * node0, 2 * _C), 2 * _C)],
                ew_v.at[b], esem.at[b])

        def wait_idx(b):
            pltpu.make_async_copy(idx_hbm.at[pl.ds(0, _G)], idx_v.at[b],
                                  isem.at[b]).wait()

        def wait_ew(b):
            pltpu.make_async_copy(ew_hbm.at[pl.ds(0, 2 * _C)], ew_v.at[b],
                                  esem.at[b]).wait()

        def issue_gather(b):
            for gg in range(_G):
                pltpu.async_copy(m_hbm.at[idx_v.at[b, gg]],
                                 rows_v.at[b, pl.ds(gg * 128, 128)],
                                 gsem.at[b])

        def wait_gather(b):
            pltpu.make_async_copy(m_hbm.at[pl.ds(0, _E)], rows_v.at[b],
                                  gsem.at[b]).wait()

        def wait_out(b):
            pltpu.make_async_copy(out_v.at[b], out_hbm.at[pl.ds(0, _E)],
                                  osem.at[b]).wait()

        def compute(g, b):
            def node_body(i, c2):
                acc0 = jnp.zeros((16,), jnp.float32)
                acc1 = jnp.zeros((16,), jnp.float32)
                for h in range(2):
                    ewv = ew_v[b, 2 * i + h]
                    for k in range(16):
                        w = ewv.at[jnp.full((16,), k, jnp.int32)].get(
                            mode="promise_in_bounds")
                        e = i * _F + h * 16 + k
                        acc0 = acc0 + w * rows_v[b, e, pl.ds(0, 16)]
                        acc1 = acc1 + w * rows_v[b, e, pl.ds(16, 16)]
                out_v[b, pl.ds(i * _F, 16)] = acc0 * (1.0 / _F)
                out_v[b, pl.ds(i * _F + 16, 16)] = acc1 * (1.0 / _F)
                return c2

            lax.fori_loop(0, _C, node_body, 0, unroll=2)
            pltpu.async_copy(
                out_v.at[b],
                out_hbm.at[pl.ds(pl.multiple_of(node0_of(g) * _F, _E), _E)],
                osem.at[b])

        issue_idx(0, 0)
        issue_ew(0, 0)
        issue_idx(1, 1)
        issue_ew(1, 1)
        wait_idx(0)
        issue_gather(0)

        def loop_body(g, carry):
            b = lax.rem(g, 2)
            nb = 1 - b
            wait_gather(b)

            @pl.when(g + 2 < _CHUNKS)
            def _():
                issue_idx(g + 2, b)

            @pl.when(g + 1 < _CHUNKS)
            def _():
                wait_idx(nb)
                issue_gather(nb)

            wait_ew(b)

            @pl.when(g >= 2)
            def _():
                wait_out(b)

            compute(g, b)

            @pl.when(g + 2 < _CHUNKS)
            def _():
                issue_ew(g + 2, b)

            return carry

        lax.fori_loop(0, _CHUNKS, loop_body, 0)
        wait_out(1)
        wait_out(0)

    return agg


def _blk4(w):
    return jnp.kron(jnp.eye(4, dtype=jnp.float32), w)


def _big384(w):
    z = jnp.zeros((384, 128), jnp.float32)
    for j in range(4):
        z = z.at[96 * j:96 * j + 32, 32 * j:32 * j + 32].set(w)
    return z


def kernel(inputs, W1p, b1p, W1u, b1u, W2p, b2p, W2u, b2u):
    inp384 = inputs.reshape(_N4, 384)

    w1p_big = _big384(W1p)
    b1p4 = jnp.tile(b1p, 4)[None, :]
    w1ua_big = _big384(W1u[:_F, :])
    w1ub = _blk4(W1u[_F:, :])
    b1u4 = jnp.tile(b1u, 4)[None, :]
    w2p = _blk4(W2p)
    b2p4 = jnp.tile(b2p, 4)[None, :]
    w2ua = _blk4(W2u[:_F, :])
    w2ub = _blk4(W2u[_F:, :])
    b2u4 = jnp.tile(b2u, 4)[None, :]

    m1_4, idx4, ewp4 = _tc_prep(inp384, w1p_big, b1p4)
    m1 = m1_4.reshape(_NPAD, _F)
    ew2 = ewp4.reshape(_NPAD * 2, 16)

    agg_fn = _make_agg()
    agg1 = agg_fn(m1, idx4, ew2)[:_N * _F].reshape(_N4, 128)
    h1_4, m2_4 = _tc_mid(inp384, agg1, w1ua_big, w1ub, b1u4, w2p, b2p4)
    agg2 = agg_fn(m2_4.reshape(_NPAD, _F), idx4, ew2)[:_N * _F].reshape(_N4, 128)
    h2_4 = _tc_final(h1_4, agg2, w2ua, w2ub, b2u4)
    return h2_4.reshape(_N, _F)


# speedup vs baseline: 1.0204x; 1.0204x over previous
import functools

import jax
import jax.numpy as jnp
from jax import lax
from jax.experimental import pallas as pl
from jax.experimental.pallas import tpu as pltpu
from jax.experimental.pallas import tpu_sc as plsc

_N = 50000
_F = 32
_N4 = _N // 4
_BLK = 512
_TCGRID = pl.cdiv(_N4, _BLK)

_NW = 32
_NPW = 1568
_NPAD = _NW * _NPW
_N4PAD = _NPAD // 4
_C = 32
_CHUNKS = _NPW // _C
_G = 8
_E = _C * _F


def _prep_body(inp_ref, wbig_ref, bp_ref, m_ref, idx_ref, ewp_ref):
    blk = inp_ref[...]
    m = jnp.maximum(
        jnp.dot(blk, wbig_ref[...], preferred_element_type=jnp.float32)
        + bp_ref[...], 0.0)
    m_ref[...] = m
    nbr = jnp.concatenate(
        [blk[:, 96 * j + 32:96 * j + 64] for j in range(4)], axis=1)
    idx_ref[...] = jnp.clip((nbr * float(_N)).astype(jnp.int32), 0, _N - 1)
    ewp_ref[...] = jnp.concatenate(
        [blk[:, 96 * j + 64:96 * j + 96] for j in range(4)], axis=1)


def _mid_body(inp_ref, a_ref, wua_ref, wub_ref, bu_ref, wp_ref, bp_ref,
              h_ref, m2_ref):
    h = jnp.maximum(
        jnp.dot(inp_ref[...], wua_ref[...], preferred_element_type=jnp.float32)
        + jnp.dot(a_ref[...], wub_ref[...], preferred_element_type=jnp.float32)
        + bu_ref[...], 0.0)
    h_ref[...] = h
    m2_ref[...] = jnp.maximum(
        jnp.dot(h, wp_ref[...], preferred_element_type=jnp.float32)
        + bp_ref[...], 0.0)


def _final_body(h_ref, a_ref, wua_ref, wub_ref, bu_ref, out_ref):
    out_ref[...] = jnp.maximum(
        jnp.dot(h_ref[...], wua_ref[...], preferred_element_type=jnp.float32)
        + jnp.dot(a_ref[...], wub_ref[...], preferred_element_type=jnp.float32)
        + bu_ref[...], 0.0)


def _row_spec():
    return pl.BlockSpec((_BLK, 128), lambda i: (i, 0))


def _inp_spec():
    return pl.BlockSpec((_BLK, 384), lambda i: (i, 0))


def _w_spec(rows=128):
    return pl.BlockSpec((rows, 128), lambda i: (0, 0))


def _b_spec():
    return pl.BlockSpec((1, 128), lambda i: (0, 0))


def _tc_prep(inp384, wbig, bp):
    return pl.pallas_call(
        _prep_body,
        grid=(_TCGRID,),
        in_specs=[_inp_spec(), _w_spec(384), _b_spec()],
        out_specs=[_row_spec(), _row_spec(), _row_spec()],
        out_shape=[
            jax.ShapeDtypeStruct((_N4PAD, 128), jnp.float32),
            jax.ShapeDtypeStruct((_N4PAD, 128), jnp.int32),
            jax.ShapeDtypeStruct((_N4PAD, 128), jnp.float32),
        ],
    )(inp384, wbig, bp)


def _tc_mid(inp384, a4, wua_big, wub, bu, wp, bp):
    return pl.pallas_call(
        _mid_body,
        grid=(_TCGRID,),
        in_specs=[_inp_spec(), _row_spec(), _w_spec(384), _w_spec(), _b_spec(),
                  _w_spec(), _b_spec()],
        out_specs=[_row_spec(), _row_spec()],
        out_shape=[
            jax.ShapeDtypeStruct((_N4, 128), jnp.float32),
            jax.ShapeDtypeStruct((_N4PAD, 128), jnp.float32),
        ],
    )(inp384, a4, wua_big, wub, bu, wp, bp)


def _tc_final(h4, a4, wua, wub, bu):
    return pl.pallas_call(
        _final_body,
        grid=(_TCGRID,),
        in_specs=[_row_spec(), _row_spec(), _w_spec(), _w_spec(), _b_spec()],
        out_specs=_row_spec(),
        out_shape=jax.ShapeDtypeStruct((_N4, 128), jnp.float32),
    )(h4, a4, wua, wub, bu)


@functools.lru_cache(maxsize=1)
def _make_agg():
    mesh = plsc.VectorSubcoreMesh(core_axis_name="c", subcore_axis_name="s")

    @functools.partial(
        pl.kernel,
        mesh=mesh,
        out_type=jax.ShapeDtypeStruct((_N4PAD, 128), jnp.float32),
        scratch_types=[
            pltpu.VMEM((2, _G, 128), jnp.int32),
            pltpu.VMEM((2, _G, 128), jnp.float32),
            pltpu.VMEM((2, _E, _F), jnp.float32),
            pltpu.VMEM((2, _G, 128), jnp.float32),
            pltpu.SemaphoreType.DMA((2,)),
            pltpu.SemaphoreType.DMA((2,)),
            pltpu.SemaphoreType.DMA((2,)),
            pltpu.SemaphoreType.DMA((2,)),
        ],
        compiler_params=pltpu.CompilerParams(use_tc_tiling_on_sc=False),
    )
    def agg(m_hbm, idx_hbm, ew_hbm, out_hbm, idx_v, ew_v, rows_v, out_v,
            isem, esem, gsem, osem):
        wid = lax.axis_index("s") * 2 + lax.axis_index("c")
        node0_w = wid * _NPW

        def node0_of(g):
            return pl.multiple_of(node0_w + g * _C, _C)

        def issue_idx(g, b):
            node0 = node0_of(g)
            row0 = pl.multiple_of(node0 // 4, _G)
            pltpu.async_copy(idx_hbm.at[pl.ds(row0, _G)], idx_v.at[b],
                             isem.at[b])

        def issue_ew(g, b):
            node0 = node0_of(g)
            row0 = pl.multiple_of(node0 // 4, _G)
            pltpu.async_copy(ew_hbm.at[pl.ds(row0, _G)], ew_v.at[b],
                             esem.at[b])

        def wait_idx(b):
            pltpu.make_async_copy(idx_hbm.at[pl.ds(0, _G)], idx_v.at[b],
                                  isem.at[b]).wait()

        def wait_ew(b):
            pltpu.make_async_copy(ew_hbm.at[pl.ds(0, _G)], ew_v.at[b],
                                  esem.at[b]).wait()

        def issue_gather(b):
            for gg in range(_G):
                pltpu.async_copy(m_hbm.at[idx_v.at[b, gg]],
                                 rows_v.at[b, pl.ds(gg * 128, 128)],
                                 gsem.at[b])

        def wait_gather(b):
            pltpu.make_async_copy(m_hbm.at[pl.ds(0, _E)], rows_v.at[b],
                                  gsem.at[b]).wait()

        def wait_out(b):
            pltpu.make_async_copy(out_v.at[b], out_hbm.at[pl.ds(0, _G)],
                                  osem.at[b]).wait()

        def compute(g, b):
            def node_body(i, c2):
                r = i // 4
                lane0 = pl.multiple_of((i % 4) * 32, 16)
                acc0 = jnp.zeros((16,), jnp.float32)
                acc1 = jnp.zeros((16,), jnp.float32)
                for h in range(2):
                    ewv = ew_v[b, r, pl.ds(pl.multiple_of(lane0 + 16 * h, 16),
                                           16)]
                    for k in range(16):
                        w = ewv.at[jnp.full((16,), k, jnp.int32)].get(
                            mode="promise_in_bounds")
                        e = i * _F + h * 16 + k
                        acc0 = acc0 + w * rows_v[b, e, pl.ds(0, 16)]
                        acc1 = acc1 + w * rows_v[b, e, pl.ds(16, 16)]
                out_v[b, r, pl.ds(lane0, 16)] = acc0
                out_v[b, r, pl.ds(pl.multiple_of(lane0 + 16, 16), 16)] = acc1
                return c2

            lax.fori_loop(0, _C, node_body, 0, unroll=2)
            pltpu.async_copy(
                out_v.at[b],
                out_hbm.at[pl.ds(pl.multiple_of(node0_of(g) // 4, _G), _G)],
                osem.at[b])

        issue_idx(0, 0)
        issue_ew(0, 0)
        issue_idx(1, 1)
        issue_ew(1, 1)
        wait_idx(0)
        issue_gather(0)

        def loop_body(g, carry):
            b = lax.rem(g, 2)
            nb = 1 - b
            wait_gather(b)

            @pl.when(g + 2 < _CHUNKS)
            def _():
                issue_idx(g + 2, b)

            @pl.when(g + 1 < _CHUNKS)
            def _():
                wait_idx(nb)
                issue_gather(nb)

            wait_ew(b)

            @pl.when(g >= 2)
            def _():
                wait_out(b)

            compute(g, b)

            @pl.when(g + 2 < _CHUNKS)
            def _():
                issue_ew(g + 2, b)

            return carry

        lax.fori_loop(0, _CHUNKS, loop_body, 0)
        wait_out(1)
        wait_out(0)

    return agg


def _blk4(w):
    return jnp.kron(jnp.eye(4, dtype=jnp.float32), w)


def _big384(w):
    z = jnp.zeros((384, 128), jnp.float32)
    for j in range(4):
        z = z.at[96 * j:96 * j + 32, 32 * j:32 * j + 32].set(w)
    return z


def kernel(inputs, W1p, b1p, W1u, b1u, W2p, b2p, W2u, b2u):
    inp384 = inputs.reshape(_N4, 384)

    w1p_big = _big384(W1p)
    b1p4 = jnp.tile(b1p, 4)[None, :]
    w1ua_big = _big384(W1u[:_F, :])
    w1ub = _blk4(W1u[_F:, :] * (1.0 / _F))
    b1u4 = jnp.tile(b1u, 4)[None, :]
    w2p = _blk4(W2p)
    b2p4 = jnp.tile(b2p, 4)[None, :]
    w2ua = _blk4(W2u[:_F, :])
    w2ub = _blk4(W2u[_F:, :] * (1.0 / _F))
    b2u4 = jnp.tile(b2u, 4)[None, :]

    m1_4, idx4, ewp4 = _tc_prep(inp384, w1p_big, b1p4)
    m1 = m1_4.reshape(_NPAD, _F)

    agg_fn = _make_agg()
    agg1 = agg_fn(m1, idx4, ewp4)
    h1_4, m2_4 = _tc_mid(inp384, agg1, w1ua_big, w1ub, b1u4, w2p, b2p4)
    agg2 = agg_fn(m2_4.reshape(_NPAD, _F), idx4, ewp4)
    h2_4 = _tc_final(h1_4, agg2, w2ua, w2ub, b2u4)
    return h2_4.reshape(_N, _F)
